# Initial kernel scaffold; baseline (speedup 1.0000x reference)
import jax
import jax.numpy as jnp
from jax.experimental import pallas as pl


def kernel(sequence_indices, table):
    raise NotImplementedError("write your pallas kernel here")


# speedup vs baseline: 2.6821x; 2.6821x over previous
import functools

import jax
import jax.numpy as jnp
from jax import lax
from jax.experimental import pallas as pl
from jax.experimental.pallas import tpu as pltpu
from jax.experimental.pallas import tpu_sc as plsc

L = 16
NC, NS = 2, 16
NW = NC * NS

BSEQ, SLEN, D = 4096, 200, 21
B = BSEQ * SLEN
PER_W = B // NW
CHUNK = 1600
NCHUNK = PER_W // CHUNK
GROUPS = CHUNK // L
CWORDS = CHUNK * D


def _make_kernel():
    mesh = plsc.VectorSubcoreMesh(core_axis_name="c", subcore_axis_name="s")

    @functools.partial(
        pl.kernel,
        mesh=mesh,
        out_type=jax.ShapeDtypeStruct((B * D,), jnp.float32),
        compiler_params=pltpu.CompilerParams(needs_layout_passes=False),
        scratch_types=[
            pltpu.VMEM((D * D,), jnp.float32),
            pltpu.VMEM((PER_W,), jnp.int32),
            pltpu.VMEM((CWORDS,), jnp.float32),
            pltpu.VMEM((CWORDS,), jnp.float32),
            pltpu.SemaphoreType.DMA,
            pltpu.SemaphoreType.DMA,
        ],
    )
    def emb(idx_hbm, table_hbm, out_hbm, table_v, idx_v, obuf0, obuf1, sem0, sem1):
        wid = lax.axis_index("s") * NC + lax.axis_index("c")
        ibase = wid * PER_W
        pltpu.sync_copy(table_hbm, table_v)
        pltpu.sync_copy(idx_hbm.at[pl.ds(ibase, PER_W)], idx_v)
        stride = lax.iota(jnp.int32, L) * D
        bufs = (obuf0, obuf1)
        sems = (sem0, sem1)
        pending = [None, None]
        for c in range(NCHUNK):
            slot = c % 2
            if pending[slot] is not None:
                pending[slot].wait()
            buf = bufs[slot]

            def body(g, _, c=c, buf=buf):
                idxv = idx_v[pl.ds(c * CHUNK + g * L, L)]
                addr = idxv * D
                sbase = stride + g * (L * D)
                for k in range(D):
                    vals = plsc.load_gather(table_v, [addr + k])
                    plsc.store_scatter(buf, [sbase + k], vals)
                return 0

            lax.fori_loop(0, GROUPS, body, 0)
            pending[slot] = pltpu.async_copy(
                buf,
                out_hbm.at[pl.ds((ibase + c * CHUNK) * D, CWORDS)],
                sems[slot],
            )
        for p in pending:
            if p is not None:
                p.wait()

    return emb


_emb = _make_kernel()


@jax.jit
def kernel(sequence_indices, table):
    idx_flat = sequence_indices.reshape(-1)
    out_flat = _emb(idx_flat, table.reshape(-1))
    return out_flat.reshape(BSEQ, SLEN, D)


# speedup vs baseline: 2.8673x; 1.0691x over previous
import functools

import jax
import jax.numpy as jnp
from jax import lax
from jax.experimental import pallas as pl
from jax.experimental.pallas import tpu as pltpu
from jax.experimental.pallas import tpu_sc as plsc

L = 16
NC, NS = 2, 16
NW = NC * NS

BSEQ, SLEN, D = 4096, 200, 21
B = BSEQ * SLEN
PER_W = B // NW
CHUNK = 1600
NCHUNK = PER_W // CHUNK
GROUPS = CHUNK // L
CWORDS = CHUNK * D


def _make_kernel():
    mesh = plsc.VectorSubcoreMesh(core_axis_name="c", subcore_axis_name="s")

    @functools.partial(
        pl.kernel,
        mesh=mesh,
        out_type=jax.ShapeDtypeStruct((B * D,), jnp.float32),
        compiler_params=pltpu.CompilerParams(needs_layout_passes=False),
        scratch_types=[
            pltpu.VMEM((D * D,), jnp.float32),
            pltpu.VMEM((PER_W,), jnp.int32),
            pltpu.VMEM((CWORDS,), jnp.float32),
            pltpu.VMEM((CWORDS,), jnp.float32),
            pltpu.SemaphoreType.DMA,
            pltpu.SemaphoreType.DMA,
        ],
    )
    def emb(idx_hbm, table_hbm, out_hbm, table_v, idx_v, obuf0, obuf1, sem0, sem1):
        wid = lax.axis_index("s") * NC + lax.axis_index("c")
        ibase = wid * PER_W
        pltpu.sync_copy(table_hbm, table_v)
        pltpu.sync_copy(idx_hbm.at[pl.ds(ibase, PER_W)], idx_v)
        stride = lax.iota(jnp.int32, L) * D
        bufs = (obuf0, obuf1)
        sems = (sem0, sem1)
        def do_chunk(c_idx, buf):
            @plsc.parallel_loop(0, GROUPS, unroll=2)
            def body(g):
                idxv = idx_v[pl.ds(c_idx * CHUNK + g * L, L)]
                addr = idxv * D
                sbase = stride + g * (L * D)
                for k in range(D):
                    vals = plsc.load_gather(table_v, [addr + k])
                    plsc.store_scatter(buf, [sbase + k], vals)

        def pair_body(t, carry):
            for slot in range(2):
                c_idx = t * 2 + slot

                @pl.when(t > 0)
                def _wait(slot=slot, c_idx=c_idx):
                    pltpu.make_async_copy(
                        bufs[slot],
                        out_hbm.at[pl.ds((ibase + (c_idx - 2) * CHUNK) * D, CWORDS)],
                        sems[slot],
                    ).wait()

                do_chunk(c_idx, bufs[slot])
                pltpu.async_copy(
                    bufs[slot],
                    out_hbm.at[pl.ds((ibase + c_idx * CHUNK) * D, CWORDS)],
                    sems[slot],
                )
            return carry

        lax.fori_loop(0, NCHUNK // 2, pair_body, 0)
        for slot in range(2):
            pltpu.make_async_copy(
                bufs[slot],
                out_hbm.at[pl.ds((ibase + (NCHUNK - 2 + slot) * CHUNK) * D, CWORDS)],
                sems[slot],
            ).wait()

    return emb


_emb = _make_kernel()


@jax.jit
def kernel(sequence_indices, table):
    idx_flat = sequence_indices.reshape(-1)
    out_flat = _emb(idx_flat, table.reshape(-1))
    return out_flat.reshape(BSEQ, SLEN, D)


# speedup vs baseline: 55.6096x; 19.3941x over previous
import functools

import jax
import jax.numpy as jnp
from jax import lax
from jax.experimental import pallas as pl
from jax.experimental.pallas import tpu as pltpu
from jax.experimental.pallas import tpu_sc as plsc

L = 16
NC, NS = 2, 16
NW = NC * NS

BSEQ, SLEN, D = 4096, 200, 21
B = BSEQ * SLEN
NBT = BSEQ // 128
NLT = SLEN // 8
TILE = 8 * 128
GROUPS = TILE // L
STAGE_W = D * TILE
NPAIR = (NLT + 2) // 2
LT_STRIDE = NBT * TILE
K_STRIDE = B


def _make_kernel():
    mesh = plsc.VectorSubcoreMesh(core_axis_name="c", subcore_axis_name="s")

    @functools.partial(
        pl.kernel,
        mesh=mesh,
        out_type=jax.ShapeDtypeStruct((B * D,), jnp.float32),
        compiler_params=pltpu.CompilerParams(needs_layout_passes=False),
        scratch_types=[
            pltpu.VMEM((D * D,), jnp.float32),
            pltpu.VMEM((NLT * TILE,), jnp.int32),
            pltpu.VMEM((STAGE_W,), jnp.float32),
            pltpu.VMEM((STAGE_W,), jnp.float32),
            pltpu.SemaphoreType.DMA,
            pltpu.SemaphoreType.DMA,
            pltpu.SemaphoreType.DMA,
        ],
    )
    def emb(idx_hbm, table_hbm, out_hbm, table_v, idx_v, stage0, stage1,
            semi, semo0, semo1):
        w = lax.axis_index("s") * NC + lax.axis_index("c")
        wbase = w * TILE
        pltpu.sync_copy(table_hbm, table_v)
        for lt in range(NLT):
            pltpu.async_copy(
                idx_hbm.at[pl.ds(lt * LT_STRIDE + wbase, TILE)],
                idx_v.at[pl.ds(lt * TILE, TILE)],
                semi,
            )
        pltpu.make_async_copy(
            idx_hbm.at[pl.ds(0, NLT * TILE)], idx_v, semi
        ).wait()

        stages = (stage0, stage1)
        semo = (semo0, semo1)

        def pair_body(t, carry):
            for slot in range(2):
                lt = t * 2 + slot

                @pl.when(lt > 1)
                def _drain(slot=slot):
                    pltpu.make_async_copy(
                        out_hbm.at[pl.ds(0, STAGE_W)], stages[slot], semo[slot]
                    ).wait()

                @pl.when(lt < NLT)
                def _work(slot=slot, lt=lt):
                    s = stages[slot]

                    @plsc.parallel_loop(0, GROUPS, unroll=4)
                    def body(g):
                        idxv = idx_v[pl.ds(lt * TILE + g * L, L)]
                        addr = idxv * D
                        for k in range(D):
                            vals = plsc.load_gather(table_v, [addr + k])
                            s[pl.ds(k * TILE + g * L, L)] = vals

                    for k in range(D):
                        pltpu.async_copy(
                            s.at[pl.ds(k * TILE, TILE)],
                            out_hbm.at[
                                pl.ds(k * K_STRIDE + lt * LT_STRIDE + wbase, TILE)
                            ],
                            semo[slot],
                        )
            return carry

        lax.fori_loop(0, NPAIR, pair_body, 0)
        pltpu.make_async_copy(
            out_hbm.at[pl.ds(0, STAGE_W)], stages[0], semo[0]
        ).wait()

    return emb


_emb = _make_kernel()


@jax.jit
def kernel(sequence_indices, table):
    idx_p = jnp.transpose(
        sequence_indices.reshape(NBT, 128, NLT, 8), (2, 0, 3, 1)
    ).reshape(-1)
    out_flat = _emb(idx_p, table.reshape(-1))
    out = jnp.transpose(
        out_flat.reshape(D, NLT, NBT, 8, 128), (2, 4, 1, 3, 0)
    ).reshape(BSEQ, SLEN, D)
    return out


# speedup vs baseline: 69.2866x; 1.2459x over previous
import functools

import jax
import jax.numpy as jnp
from jax import lax
from jax.experimental import pallas as pl
from jax.experimental.pallas import tpu as pltpu
from jax.experimental.pallas import tpu_sc as plsc

L = 16
NC, NS = 2, 16
NW = NC * NS

BSEQ, SLEN, D = 4096, 200, 21
B = BSEQ * SLEN
NROW = B // 128
PER_W = B // NW
TILE = 8 * 128
NTASK = PER_W // TILE
GROUPS = TILE // L
NPAIR = (NTASK + 2) // 2


def _make_kernel():
    mesh = plsc.VectorSubcoreMesh(core_axis_name="c", subcore_axis_name="s")

    @functools.partial(
        pl.kernel,
        mesh=mesh,
        out_type=jax.ShapeDtypeStruct((D, NROW, 128), jnp.float32),
        compiler_params=pltpu.CompilerParams(needs_layout_passes=False),
        scratch_types=[
            pltpu.VMEM((D * D,), jnp.float32),
            pltpu.VMEM((PER_W,), jnp.int32),
            pltpu.VMEM((D, 8, 128), jnp.float32),
            pltpu.VMEM((D, 8, 128), jnp.float32),
            pltpu.SemaphoreType.DMA,
            pltpu.SemaphoreType.DMA,
        ],
    )
    def emb(idx_hbm, table_hbm, out_hbm, table_v, idx_v, stage0, stage1,
            semo0, semo1):
        w = lax.axis_index("s") * NC + lax.axis_index("c")
        rbase = w * (PER_W // 128)
        pltpu.sync_copy(table_hbm, table_v)
        pltpu.sync_copy(idx_hbm.at[pl.ds(w * PER_W, PER_W)], idx_v)

        stages = (stage0, stage1)
        semo = (semo0, semo1)

        def pair_body(p, carry):
            for slot in range(2):
                t = p * 2 + slot

                @pl.when(t > 1)
                def _drain(slot=slot):
                    pltpu.make_async_copy(
                        out_hbm.at[:, pl.ds(0, 8), :], stages[slot], semo[slot]
                    ).wait()

                @pl.when(t < NTASK)
                def _work(slot=slot, t=t):
                    s = stages[slot]

                    @plsc.parallel_loop(0, GROUPS, unroll=2)
                    def body(g):
                        idxv = idx_v[pl.ds(t * TILE + g * L, L)]
                        addr = idxv * D
                        r = lax.shift_right_logical(g, 3)
                        c0 = lax.bitwise_and(g, 7) * L
                        for k in range(D):
                            vals = plsc.load_gather(table_v, [addr + k])
                            s[k, r, pl.ds(c0, L)] = vals

                    pltpu.async_copy(
                        s,
                        out_hbm.at[:, pl.ds(rbase + t * 8, 8), :],
                        semo[slot],
                    )
            return carry

        lax.fori_loop(0, NPAIR, pair_body, 0)
        pltpu.make_async_copy(
            out_hbm.at[:, pl.ds(0, 8), :], stages[0], semo[0]
        ).wait()

    return emb


_emb = _make_kernel()


@jax.jit
def kernel(sequence_indices, table):
    NBT, NLT = BSEQ // 128, SLEN // 8
    idx_p = jnp.transpose(
        sequence_indices.reshape(NBT, 128, NLT, 8), (2, 0, 3, 1)
    ).reshape(-1)
    out_t = _emb(idx_p, table.reshape(-1))
    out = jnp.transpose(
        out_t.reshape(D, NLT, NBT, 8, 128), (2, 4, 1, 3, 0)
    ).reshape(BSEQ, SLEN, D)
    return out
